# baseline (speedup 1.0000x reference)
import functools
import jax
import jax.numpy as jnp
from jax import lax
from jax.experimental import pallas as pl
from jax.experimental.pallas import tpu as pltpu
from jax.experimental.pallas import tpu_sc as plsc

NC = 2
NS = 16
LANES = 16
NW = NC * NS
NBUF = 3


def _emb_call(ids_flat, token_table, pos_table, *, batch, seq, chunk):
    d = token_table.shape[1]
    d_vecs = d // LANES
    ppw = seq // NW
    n_pchunks = ppw // chunk
    n_tasks = n_pchunks * batch

    mesh = plsc.VectorSubcoreMesh(core_axis_name="c", subcore_axis_name="s")

    @functools.partial(
        pl.kernel,
        out_type=jax.ShapeDtypeStruct((batch * seq, d), jnp.float32),
        mesh=mesh,
        scratch_types=[
            pltpu.VMEM((batch, ppw), jnp.int32),
            [pltpu.VMEM((chunk, d), jnp.float32) for _ in range(NBUF)],
            [pltpu.VMEM((chunk, d), jnp.float32) for _ in range(2)],
            [pltpu.SemaphoreType.DMA for _ in range(NBUF)],
            [pltpu.SemaphoreType.DMA for _ in range(2)],
            [pltpu.SemaphoreType.DMA for _ in range(NBUF)],
        ],
    )
    def k(ids_hbm, tok_hbm, pos_hbm, out_hbm, idx_v, rows, pos, gsem, psem, osem):
        wid = lax.axis_index("s") * NC + lax.axis_index("c")
        wpos = wid * ppw
        store_h = [None] * NBUF
        pos_h = [None, None]
        gather_h = {}

        for b in range(batch):
            pltpu.sync_copy(ids_hbm.at[pl.ds(b * seq + wpos, ppw)], idx_v.at[b])

        def row_off(t):
            p, b = t // batch, t % batch
            return b * seq + wpos + p * chunk

        def start_pos(p):
            pb = p & 1
            pos_h[pb] = pltpu.async_copy(
                pos_hbm.at[pl.ds(wpos + p * chunk, chunk), :], pos[pb], psem[pb])

        def start_gather(t):
            r = t % NBUF
            if store_h[r] is not None:
                store_h[r].wait()
            p, b = t // batch, t % batch
            gather_h[t] = pltpu.async_copy(
                tok_hbm.at[idx_v.at[b, pl.ds(p * chunk, chunk)]], rows[r], gsem[r])

        start_pos(0)
        for t in range(min(NBUF - 1, n_tasks)):
            start_gather(t)
        for t in range(n_tasks):
            r = t % NBUF
            p, b = t // batch, t % batch
            if t + NBUF - 1 < n_tasks:
                start_gather(t + NBUF - 1)
            if b == 0:
                pos_h[p & 1].wait()
                if p + 1 < n_pchunks:
                    start_pos(p + 1)
            gather_h.pop(t).wait()
            pbuf = pos[p & 1]

            def row_body(i, _):
                for j in range(d_vecs):
                    sl = pl.ds(j * LANES, LANES)
                    plsc.addupdate(rows[r].at[i, sl], pbuf[i, sl])
                return 0

            lax.fori_loop(0, chunk, row_body, 0)
            store_h[r] = pltpu.async_copy(
                rows[r], out_hbm.at[pl.ds(row_off(t), chunk), :], osem[r])
        for h in store_h:
            if h is not None:
                h.wait()

    return k(ids_flat, token_table, pos_table)


def kernel(input_ids, token_table, pos_table):
    b, s = input_ids.shape
    d = token_table.shape[1]
    ids_flat = input_ids.reshape(-1).astype(jnp.int32)
    out = _emb_call(ids_flat, token_table, pos_table, batch=b, seq=s, chunk=32)
    return out.reshape(b, s, d)


# speedup vs baseline: 1.0159x; 1.0159x over previous
import functools
import jax
import jax.numpy as jnp
from jax import lax
from jax.experimental import pallas as pl
from jax.experimental.pallas import tpu as pltpu
from jax.experimental.pallas import tpu_sc as plsc

NC = 2
NS = 16
LANES = 16
NW = NC * NS
NBUF = 3


def _emb_call(ids_flat, token_table, pos_table, *, batch, seq, chunk):
    d = token_table.shape[1]
    d_vecs = d // LANES
    ppw = seq // NW
    n_pchunks = ppw // chunk
    n_tasks = n_pchunks * batch

    mesh = plsc.VectorSubcoreMesh(core_axis_name="c", subcore_axis_name="s")

    @functools.partial(
        pl.kernel,
        out_type=jax.ShapeDtypeStruct((batch * seq, d), jnp.float32),
        mesh=mesh,
        scratch_types=[
            pltpu.VMEM((batch, ppw), jnp.int32),
            [pltpu.VMEM((chunk, d), jnp.float32) for _ in range(NBUF)],
            [pltpu.VMEM((chunk, d), jnp.float32) for _ in range(2)],
            [pltpu.SemaphoreType.DMA for _ in range(NBUF)],
            [pltpu.SemaphoreType.DMA for _ in range(2)],
            [pltpu.SemaphoreType.DMA for _ in range(NBUF)],
        ],
    )
    def k(ids_hbm, tok_hbm, pos_hbm, out_hbm, idx_v, rows, pos, gsem, psem, osem):
        wid = lax.axis_index("s") * NC + lax.axis_index("c")
        wpos = wid * ppw
        store_h = [None] * NBUF
        pos_h = [None, None]
        gather_h = {}

        def row_off(t):
            p, b = t // batch, t % batch
            return b * seq + wpos + p * chunk

        def start_pos(p):
            pb = p & 1
            pos_h[pb] = pltpu.async_copy(
                pos_hbm.at[pl.ds(wpos + p * chunk, chunk), :], pos[pb], psem[pb])

        def start_gather(t):
            r = t % NBUF
            if store_h[r] is not None:
                store_h[r].wait()
            p, b = t // batch, t % batch
            gather_h[t] = pltpu.async_copy(
                tok_hbm.at[idx_v.at[b, pl.ds(p * chunk, chunk)]], rows[r], gsem[r])

        start_pos(0)
        idx_h = [pltpu.async_copy(ids_hbm.at[pl.ds(b * seq + wpos, ppw)],
                                  idx_v.at[b], osem[0])
                 for b in range(batch)]
        for h in idx_h:
            h.wait()
        for t in range(min(NBUF - 1, n_tasks)):
            start_gather(t)
        for t in range(n_tasks):
            r = t % NBUF
            p, b = t // batch, t % batch
            if t + NBUF - 1 < n_tasks:
                start_gather(t + NBUF - 1)
            if b == 0:
                pos_h[p & 1].wait()
                if p + 1 < n_pchunks:
                    start_pos(p + 1)
            gather_h.pop(t).wait()
            pbuf = pos[p & 1]

            def row_body(i, _):
                for j in range(d_vecs):
                    sl = pl.ds(j * LANES, LANES)
                    plsc.addupdate(rows[r].at[i, sl], pbuf[i, sl])
                return 0

            lax.fori_loop(0, chunk, row_body, 0)
            store_h[r] = pltpu.async_copy(
                rows[r], out_hbm.at[pl.ds(row_off(t), chunk), :], osem[r])
        for h in store_h:
            if h is not None:
                h.wait()

    return k(ids_flat, token_table, pos_table)


def kernel(input_ids, token_table, pos_table):
    b, s = input_ids.shape
    d = token_table.shape[1]
    ids_flat = input_ids.reshape(-1).astype(jnp.int32)
    out = _emb_call(ids_flat, token_table, pos_table, batch=b, seq=s, chunk=32)
    return out.reshape(b, s, d)
